# Initial kernel scaffold; baseline (speedup 1.0000x reference)
import jax
import jax.numpy as jnp
from jax.experimental import pallas as pl


def kernel(input, table):
    raise NotImplementedError("write your pallas kernel here")


# speedup vs baseline: 1.8378x; 1.8378x over previous
import functools

import jax
import jax.numpy as jnp
from jax import lax
from jax.experimental import pallas as pl
from jax.experimental.pallas import tpu as pltpu
from jax.experimental.pallas import tpu_sc as plsc


def _embed_call(NW, NC, n_chunks, C, D, N):
    mesh = plsc.VectorSubcoreMesh(core_axis_name="c", subcore_axis_name="s")
    per_w = n_chunks * C

    @functools.partial(
        pl.kernel,
        mesh=mesh,
        compiler_params=pltpu.CompilerParams(use_tc_tiling_on_sc=False),
        out_type=jax.ShapeDtypeStruct((N, D), jnp.float32),
        scratch_types=[
            pltpu.VMEM((n_chunks, C), jnp.int32),
            pltpu.VMEM((2, C, D), jnp.float32),
            pltpu.SemaphoreType.DMA,
            pltpu.SemaphoreType.DMA,
        ],
    )
    def k(idx_hbm, table_hbm, out_hbm, idx_v, rows_v, sem0, sem1):
        wid = lax.axis_index("s") * NC + lax.axis_index("c")
        base = wid * per_w
        pltpu.sync_copy(idx_hbm.at[wid], idx_v)
        sems = (sem0, sem1)

        pltpu.async_copy(table_hbm.at[idx_v.at[0]], rows_v.at[0], sem0)

        def pair(g, carry):
            for b in range(2):
                j = 2 * g + b
                nb = (b + 1) % 2

                @pl.when(j + 1 < n_chunks)
                def _():
                    pltpu.async_copy(
                        table_hbm.at[idx_v.at[j + 1]], rows_v.at[nb], sems[nb]
                    )

                pltpu.make_async_copy(
                    table_hbm.at[pl.ds(0, C)], rows_v.at[b], sems[b]
                ).wait()
                pltpu.sync_copy(rows_v.at[b], out_hbm.at[pl.ds(base + j * C, C)])
            return carry

        lax.fori_loop(0, n_chunks // 2, pair, 0)

    return k


def kernel(input, table):
    B0, S = input.shape
    V, D = table.shape
    flat = input.reshape(-1).astype(jnp.int32)
    N = flat.shape[0]

    info = plsc.get_sparse_core_info()
    NC, NS = info.num_cores, info.num_subcores
    NW = NC * NS
    C = 128
    n_chunks = N // (NW * C)

    idx3 = flat.reshape(NW, n_chunks, C)
    out = _embed_call(NW, NC, n_chunks, C, D, N)(idx3, table)
    return out.reshape(B0, S, D)
